# baseline (speedup 1.0000x reference)
import functools

import jax
import jax.numpy as jnp
from jax import lax
from jax.experimental import pallas as pl
from jax.experimental.pallas import tpu as pltpu
from jax.experimental.pallas import tpu_sc as plsc

N = 10000
F_IN = 256
H = 64
E = 160000

NC = 2
NS = 16
NW = NC * NS
CH = 128
BUF = E + NW * CH
RPW = 313
NPAD = NW * RPW
EPW = E // NW
DCH = 125
DNCH = EPW // DCH
RPT = 624
TAIL = N - NS * RPT
DEGW = 16

_sc_mesh = plsc.VectorSubcoreMesh(
    core_axis_name="c", subcore_axis_name="s", num_cores=NC, num_subcores=NS)
_sc_params = pltpu.CompilerParams(use_tc_tiling_on_sc=False,
                                 needs_layout_passes=False)


@functools.partial(
    pl.kernel,
    mesh=_sc_mesh,
    out_type=jax.ShapeDtypeStruct((NC, N, DEGW), jnp.float32),
    compiler_params=_sc_params,
    scratch_types=[
        pltpu.VMEM((DNCH, DCH), jnp.int32),
        pltpu.VMEM((DCH, DEGW), jnp.float32),
        pltpu.VMEM_SHARED((N, DEGW), jnp.float32),
    ],
)
def _sc_degree(dst_hbm, ones_hbm, zeros_hbm, out_hbm, dst_v, ones_v, deg_sh):
    cid = lax.axis_index("c")
    sid = lax.axis_index("s")
    wid = sid * NC + cid
    pltpu.sync_copy(dst_hbm.at[wid], dst_v)
    pltpu.sync_copy(ones_hbm, ones_v)
    pltpu.sync_copy(zeros_hbm, deg_sh.at[pl.ds(sid * RPT, RPT)])

    @pl.when(sid == 0)
    def _():
        pltpu.sync_copy(zeros_hbm.at[pl.ds(0, TAIL)],
                        deg_sh.at[pl.ds(NS * RPT, TAIL)])

    plsc.subcore_barrier()

    def body(j, carry):
        pltpu.sync_copy(ones_v, deg_sh.at[dst_v.at[j]], add=True)
        return carry

    lax.fori_loop(0, DNCH, body, 0)
    plsc.subcore_barrier()
    pltpu.sync_copy(deg_sh.at[pl.ds(sid * RPT, RPT)],
                    out_hbm.at[cid, pl.ds(sid * RPT, RPT)])

    @pl.when(sid == 0)
    def _():
        pltpu.sync_copy(deg_sh.at[pl.ds(NS * RPT, TAIL)],
                        out_hbm.at[cid, pl.ds(NS * RPT, TAIL)])


@functools.partial(
    pl.kernel,
    mesh=_sc_mesh,
    out_type=jax.ShapeDtypeStruct((NW, RPW, H), jnp.float32),
    compiler_params=_sc_params,
    scratch_types=[
        pltpu.VMEM((32,), jnp.int32),
        pltpu.VMEM((32,), jnp.int32),
        pltpu.VMEM((CH,), jnp.int32),
        pltpu.VMEM((CH,), jnp.int32),
        pltpu.VMEM((CH,), jnp.float32),
        pltpu.VMEM((CH,), jnp.int32),
        pltpu.VMEM((CH, H), jnp.float32),
        pltpu.VMEM((NPAD, 1), jnp.float32),
        pltpu.VMEM((RPW + 1, H), jnp.float32),
        pltpu.SemaphoreType.DMA,
    ],
)
def _sc_aggregate(src_hbm, dst_hbm, nch_hbm, offs_hbm, xw_hbm, dinv_hbm,
                  zeros_hbm, out_hbm, nch_v, offs_v, src_v, dst_v, norm_v,
                  dloc_v, rows_v, dinv_v, agg_v, sem):
    cid = lax.axis_index("c")
    sid = lax.axis_index("s")
    wid = sid * NC + cid
    pltpu.sync_copy(nch_hbm, nch_v)
    pltpu.sync_copy(offs_hbm, offs_v)
    pltpu.sync_copy(dinv_hbm, dinv_v)
    pltpu.sync_copy(zeros_hbm, agg_v)
    widv = jnp.full((16,), wid, dtype=jnp.int32)
    nch = jnp.max(plsc.load_gather(nch_v, [widv]))
    off = jnp.max(plsc.load_gather(offs_v, [widv]))
    row0 = wid * RPW
    zeros16 = jnp.zeros((16,), jnp.int32)
    iota16 = lax.iota(jnp.int32, 16)

    def chunk_body(j, carry):
        base = (off + j) * CH
        pltpu.sync_copy(src_hbm.at[pl.ds(base, CH)], src_v)
        pltpu.sync_copy(dst_hbm.at[pl.ds(base, CH)], dst_v)
        pltpu.async_copy(xw_hbm.at[src_v], rows_v, sem).wait()

        def norm_body(g, c2):
            sg = src_v[pl.ds(g * 16, 16)]
            dg = dst_v[pl.ds(g * 16, 16)]
            nv = (plsc.load_gather(dinv_v, [sg, zeros16])
                  * plsc.load_gather(dinv_v, [dg, zeros16]))
            norm_v[pl.ds(g * 16, 16)] = nv
            dloc_v[pl.ds(g * 16, 16)] = dg - row0
            return c2

        lax.fori_loop(0, CH // 16, norm_body, 0)

        def edge_body(eb, c3):
            for u in range(4):
                e = eb * 4 + u
                ev = jnp.full((16,), e, dtype=jnp.int32)
                ne = plsc.load_gather(norm_v, [ev])
                re = plsc.load_gather(dloc_v, [ev])
                for q in range(H // 16):
                    msg = rows_v[e, pl.ds(q * 16, 16)] * ne
                    plsc.addupdate_scatter(agg_v, [re, iota16 + (q * 16)], msg)
            return c3

        lax.fori_loop(0, CH // 4, edge_body, 0)
        return carry

    lax.fori_loop(0, nch, chunk_body, 0)
    pltpu.sync_copy(agg_v.at[pl.ds(0, RPW)], out_hbm.at[wid])


BN = 1000


def _dinv_block(deg_ref):
    deg = deg_ref[0, :, 0:1] + deg_ref[1, :, 0:1] + 1.0
    return 1.0 / jnp.sqrt(deg)


def _tc_xw1_body(deg_ref, x_ref, w_ref, xw_ref, dinv_ref):
    dinv_ref[...] = _dinv_block(deg_ref)
    xw_ref[...] = jnp.dot(x_ref[...], w_ref[...],
                          preferred_element_type=jnp.float32)


def _tc_layer2_body(deg_ref, agg_ref, xw_ref, b_ref, w_ref, xw2_ref):
    dinv = _dinv_block(deg_ref)
    h = agg_ref[...] + xw_ref[...] * (dinv * dinv) + b_ref[...]
    h = jnp.maximum(h, 0.0)
    xw2_ref[...] = jnp.dot(h, w_ref[...], preferred_element_type=jnp.float32)


def _tc_h2_body(deg_ref, agg_ref, xw_ref, b_ref, h_ref):
    dinv = _dinv_block(deg_ref)
    h = agg_ref[...] + xw_ref[...] * (dinv * dinv) + b_ref[...]
    h_ref[...] = jnp.maximum(h, 0.0)


BK = 16000
NKB = (N * H) // BK


def _tc_head_body(h_ref, w1_ref, b1_ref, w2_ref, b2_ref, w3_ref, b3_ref,
                  out_ref, acc_ref):
    k = pl.program_id(0)

    @pl.when(k == 0)
    def _():
        acc_ref[...] = jnp.zeros_like(acc_ref)

    acc_ref[...] += jnp.dot(h_ref[...], w1_ref[...],
                            preferred_element_type=jnp.float32)

    @pl.when(k == NKB - 1)
    def _():
        z1 = jnp.maximum(acc_ref[...] + b1_ref[...], 0.0)
        z2 = jnp.maximum(
            jnp.dot(z1, w2_ref[...], preferred_element_type=jnp.float32)
            + b2_ref[...], 0.0)
        out_ref[...] = jnp.dot(
            z2, w3_ref[...], preferred_element_type=jnp.float32,
            precision=lax.Precision.HIGHEST) + b3_ref[...]


def kernel(x, edge_index, W1, b1, W2, b2, fW1, fb1, fW2, fb2, fW3, fb3):
    f32 = jnp.float32
    i32 = jnp.int32
    src0 = edge_index[0]
    dst0 = edge_index[1]

    b = dst0 // RPW
    perm = jnp.argsort(b, stable=True)
    ss = src0[perm]
    ds_ = dst0[perm]
    bs = b[perm]
    counts = jnp.zeros((NW,), i32).at[b].add(1)
    cpad = ((counts + (CH - 1)) // CH) * CH
    offs_e = jnp.concatenate([jnp.zeros((1,), i32), jnp.cumsum(cpad)[:-1]])
    starts = jnp.concatenate([jnp.zeros((1,), i32), jnp.cumsum(counts)[:-1]])
    pos = offs_e[bs] + jnp.arange(E, dtype=i32) - starts[bs]
    rid = jnp.searchsorted(offs_e, jnp.arange(BUF, dtype=i32), side='right') - 1
    src_buf = jnp.full((BUF,), N, i32).at[pos].set(ss)
    dst_buf = (rid.astype(i32) * RPW).at[pos].set(ds_)
    nch = (cpad // CH).astype(i32)
    offs = (offs_e // CH).astype(i32)

    dst_deg = dst0.reshape(NW, DNCH, DCH)
    zeros_deg = jnp.zeros((RPT, DEGW), f32)
    ones_deg = jnp.ones((DCH, DEGW), f32)
    deg_parts = _sc_degree(dst_deg, ones_deg, zeros_deg)

    grid_n = N // BN
    deg_spec = pl.BlockSpec((NC, BN, DEGW), lambda i: (0, i, 0))
    row_h = pl.BlockSpec((BN, H), lambda i: (i, 0))
    full = lambda *shape: pl.BlockSpec(shape, lambda i: (0,) * len(shape))

    xw1, dinv = pl.pallas_call(
        _tc_xw1_body,
        grid=(grid_n,),
        in_specs=[deg_spec,
                  pl.BlockSpec((BN, F_IN), lambda i: (i, 0)),
                  full(F_IN, H)],
        out_specs=[row_h, pl.BlockSpec((BN, 1), lambda i: (i, 0))],
        out_shape=[jax.ShapeDtypeStruct((N, H), f32),
                   jax.ShapeDtypeStruct((N, 1), f32)],
    )(deg_parts, x, W1)

    pad_h = jnp.zeros((NPAD - N, H), f32)
    dinv_p = jnp.concatenate([dinv, jnp.zeros((NPAD - N, 1), f32)])
    zeros_agg = jnp.zeros((RPW + 1, H), f32)

    xw1_p = jnp.concatenate([xw1, pad_h])
    agg1 = _sc_aggregate(src_buf, dst_buf, nch, offs, xw1_p, dinv_p,
                         zeros_agg).reshape(NPAD, H)

    agg_spec = pl.BlockSpec((BN, H), lambda i: (i, 0))
    xw2 = pl.pallas_call(
        _tc_layer2_body,
        grid=(grid_n,),
        in_specs=[deg_spec, agg_spec, row_h, full(1, H), full(H, H)],
        out_specs=row_h,
        out_shape=jax.ShapeDtypeStruct((N, H), f32),
    )(deg_parts, agg1, xw1, b1.reshape(1, H), W2)

    xw2_p = jnp.concatenate([xw2, pad_h])
    agg2 = _sc_aggregate(src_buf, dst_buf, nch, offs, xw2_p, dinv_p,
                         zeros_agg).reshape(NPAD, H)

    h2 = pl.pallas_call(
        _tc_h2_body,
        grid=(grid_n,),
        in_specs=[deg_spec, agg_spec, row_h, full(1, H)],
        out_specs=row_h,
        out_shape=jax.ShapeDtypeStruct((N, H), f32),
    )(deg_parts, agg2, xw2, b2.reshape(1, H))

    out = pl.pallas_call(
        _tc_head_body,
        grid=(NKB,),
        in_specs=[pl.BlockSpec((1, BK), lambda k: (0, k)),
                  pl.BlockSpec((BK, 128), lambda k: (k, 0)),
                  full(1, 128), full(128, 64), full(1, 64),
                  full(64, 1), full(1, 1)],
        out_specs=pl.BlockSpec((1, 1), lambda k: (0, 0)),
        out_shape=jax.ShapeDtypeStruct((1, 1), f32),
        scratch_shapes=[pltpu.VMEM((1, 128), f32)],
        compiler_params=pltpu.CompilerParams(
            dimension_semantics=("arbitrary",)),
    )(h2.reshape(1, N * H), fW1, fb1.reshape(1, 128), fW2,
      fb2.reshape(1, 64), fW3, fb3.reshape(1, 1))

    return out


# speedup vs baseline: 6.9919x; 6.9919x over previous
import functools

import jax
import jax.numpy as jnp
from jax import lax
from jax.experimental import pallas as pl
from jax.experimental.pallas import tpu as pltpu
from jax.experimental.pallas import tpu_sc as plsc

N = 10000
F_IN = 256
H = 64
E = 160000

NC = 2
NS = 16
NW = NC * NS
EPW = E // NW
CH = 125
NCHUNK = EPW // CH
RPT = 624
TAIL = N - NS * RPT
DEGW = 16

_sc_mesh = plsc.VectorSubcoreMesh(
    core_axis_name="c", subcore_axis_name="s", num_cores=NC, num_subcores=NS)
_sc_params = pltpu.CompilerParams(use_tc_tiling_on_sc=False)


@functools.partial(
    pl.kernel,
    mesh=_sc_mesh,
    out_type=jax.ShapeDtypeStruct((NC, N, DEGW), jnp.float32),
    compiler_params=_sc_params,
    scratch_types=[
        pltpu.VMEM((NCHUNK, CH), jnp.int32),
        pltpu.VMEM((CH, DEGW), jnp.float32),
        pltpu.VMEM_SHARED((N, DEGW), jnp.float32),
    ],
)
def _sc_degree(dst_hbm, ones_hbm, zeros_hbm, out_hbm, dst_v, ones_v, deg_sh):
    cid = lax.axis_index("c")
    sid = lax.axis_index("s")
    wid = sid * NC + cid
    pltpu.sync_copy(dst_hbm.at[wid], dst_v)
    pltpu.sync_copy(ones_hbm, ones_v)
    pltpu.sync_copy(zeros_hbm, deg_sh.at[pl.ds(sid * RPT, RPT)])

    @pl.when(sid == 0)
    def _():
        pltpu.sync_copy(zeros_hbm.at[pl.ds(0, TAIL)],
                        deg_sh.at[pl.ds(NS * RPT, TAIL)])

    plsc.subcore_barrier()

    def body(j, carry):
        pltpu.sync_copy(ones_v, deg_sh.at[dst_v.at[j]], add=True)
        return carry

    lax.fori_loop(0, NCHUNK, body, 0)
    plsc.subcore_barrier()
    pltpu.sync_copy(deg_sh.at[pl.ds(sid * RPT, RPT)],
                    out_hbm.at[cid, pl.ds(sid * RPT, RPT)])

    @pl.when(sid == 0)
    def _():
        pltpu.sync_copy(deg_sh.at[pl.ds(NS * RPT, TAIL)],
                        out_hbm.at[cid, pl.ds(NS * RPT, TAIL)])


@functools.partial(
    pl.kernel,
    mesh=_sc_mesh,
    out_type=jax.ShapeDtypeStruct((NC, N, H), jnp.float32),
    compiler_params=_sc_params,
    scratch_types=[
        pltpu.VMEM((NCHUNK, CH), jnp.int32),
        pltpu.VMEM((NCHUNK, CH), jnp.int32),
        pltpu.VMEM((CH, H), jnp.float32),
        pltpu.VMEM_SHARED((N, H), jnp.float32),
        pltpu.SemaphoreType.DMA,
    ],
)
def _sc_aggregate(src_hbm, dst_hbm, y_hbm, zeros_hbm, out_hbm,
                  src_v, dst_v, rows_v, agg_sh, sem):
    cid = lax.axis_index("c")
    sid = lax.axis_index("s")
    wid = sid * NC + cid
    pltpu.sync_copy(src_hbm.at[wid], src_v)
    pltpu.sync_copy(dst_hbm.at[wid], dst_v)
    pltpu.sync_copy(zeros_hbm, agg_sh.at[pl.ds(sid * RPT, RPT)])

    @pl.when(sid == 0)
    def _():
        pltpu.sync_copy(zeros_hbm.at[pl.ds(0, TAIL)],
                        agg_sh.at[pl.ds(NS * RPT, TAIL)])

    plsc.subcore_barrier()

    def body(j, carry):
        pltpu.async_copy(y_hbm.at[src_v.at[j]], rows_v, sem).wait()
        pltpu.sync_copy(rows_v, agg_sh.at[dst_v.at[j]], add=True)
        return carry

    lax.fori_loop(0, NCHUNK, body, 0)
    plsc.subcore_barrier()
    pltpu.sync_copy(agg_sh.at[pl.ds(sid * RPT, RPT)],
                    out_hbm.at[cid, pl.ds(sid * RPT, RPT)])

    @pl.when(sid == 0)
    def _():
        pltpu.sync_copy(agg_sh.at[pl.ds(NS * RPT, TAIL)],
                        out_hbm.at[cid, pl.ds(NS * RPT, TAIL)])


BN = 1000


def _dinv_block(deg_ref):
    deg = deg_ref[0, :, 0:1] + deg_ref[1, :, 0:1] + 1.0
    return lax.rsqrt(deg)


def _tc_scale1_body(deg_ref, x_ref, w_ref, y_ref):
    dinv = _dinv_block(deg_ref)
    xw = jnp.dot(x_ref[...], w_ref[...], preferred_element_type=jnp.float32)
    y_ref[...] = xw * dinv


def _tc_layer2_body(deg_ref, agg_ref, y1_ref, b_ref, w_ref, y2_ref):
    dinv = _dinv_block(deg_ref)
    h = agg_ref[0] + agg_ref[1] + y1_ref[...]
    h = jnp.maximum(h * dinv + b_ref[...], 0.0)
    y2_ref[...] = jnp.dot(h, w_ref[...],
                          preferred_element_type=jnp.float32) * dinv


def _tc_h2_body(deg_ref, agg_ref, y2_ref, b_ref, h_ref):
    dinv = _dinv_block(deg_ref)
    h = agg_ref[0] + agg_ref[1] + y2_ref[...]
    h_ref[...] = jnp.maximum(h * dinv + b_ref[...], 0.0)


BK = 16000
NKB = (N * H) // BK


def _tc_head_body(h_ref, w1_ref, b1_ref, w2_ref, b2_ref, w3_ref, b3_ref,
                  out_ref, acc_ref):
    k = pl.program_id(0)

    @pl.when(k == 0)
    def _():
        acc_ref[...] = jnp.zeros_like(acc_ref)

    acc_ref[...] += jnp.dot(h_ref[...], w1_ref[...],
                            preferred_element_type=jnp.float32)

    @pl.when(k == NKB - 1)
    def _():
        z1 = jnp.maximum(acc_ref[...] + b1_ref[...], 0.0)
        z2 = jnp.maximum(
            jnp.dot(z1, w2_ref[...], preferred_element_type=jnp.float32)
            + b2_ref[...], 0.0)
        out_ref[...] = jnp.dot(
            z2, w3_ref[...], preferred_element_type=jnp.float32,
            precision=lax.Precision.HIGHEST) + b3_ref[...]


def kernel(x, edge_index, W1, b1, W2, b2, fW1, fb1, fW2, fb2, fW3, fb3):
    f32 = jnp.float32
    src = edge_index[0].reshape(NW, NCHUNK, CH)
    dst = edge_index[1].reshape(NW, NCHUNK, CH)
    zeros_h = jnp.zeros((RPT, H), f32)
    zeros_d = jnp.zeros((RPT, DEGW), f32)
    ones_d = jnp.ones((CH, DEGW), f32)

    deg_parts = _sc_degree(dst, ones_d, zeros_d)

    grid_n = N // BN
    deg_spec = pl.BlockSpec((NC, BN, DEGW), lambda i: (0, i, 0))
    row_h = pl.BlockSpec((BN, H), lambda i: (i, 0))
    agg_spec = pl.BlockSpec((NC, BN, H), lambda i: (0, i, 0))
    full = lambda *shape: pl.BlockSpec(shape, lambda i: (0,) * len(shape))

    y1 = pl.pallas_call(
        _tc_scale1_body,
        grid=(grid_n,),
        in_specs=[deg_spec,
                  pl.BlockSpec((BN, F_IN), lambda i: (i, 0)),
                  full(F_IN, H)],
        out_specs=row_h,
        out_shape=jax.ShapeDtypeStruct((N, H), f32),
    )(deg_parts, x, W1)

    agg1 = _sc_aggregate(src, dst, y1, zeros_h)

    y2 = pl.pallas_call(
        _tc_layer2_body,
        grid=(grid_n,),
        in_specs=[deg_spec, agg_spec, row_h, full(1, H), full(H, H)],
        out_specs=row_h,
        out_shape=jax.ShapeDtypeStruct((N, H), f32),
    )(deg_parts, agg1, y1, b1.reshape(1, H), W2)

    agg2 = _sc_aggregate(src, dst, y2, zeros_h)

    h2 = pl.pallas_call(
        _tc_h2_body,
        grid=(grid_n,),
        in_specs=[deg_spec, agg_spec, row_h, full(1, H)],
        out_specs=row_h,
        out_shape=jax.ShapeDtypeStruct((N, H), f32),
    )(deg_parts, agg2, y2, b2.reshape(1, H))

    out = pl.pallas_call(
        _tc_head_body,
        grid=(NKB,),
        in_specs=[pl.BlockSpec((1, BK), lambda k: (0, k)),
                  pl.BlockSpec((BK, 128), lambda k: (k, 0)),
                  full(1, 128), full(128, 64), full(1, 64),
                  full(64, 1), full(1, 1)],
        out_specs=pl.BlockSpec((1, 1), lambda k: (0, 0)),
        out_shape=jax.ShapeDtypeStruct((1, 1), f32),
        scratch_shapes=[pltpu.VMEM((1, 128), f32)],
        compiler_params=pltpu.CompilerParams(
            dimension_semantics=("arbitrary",)),
    )(h2.reshape(1, N * H), fW1, fb1.reshape(1, 128), fW2,
      fb2.reshape(1, 64), fW3, fb3.reshape(1, 1))

    return out
